# baseline (speedup 1.0000x reference)
import functools

import jax
import jax.numpy as jnp
from jax import lax
from jax.experimental import pallas as pl
from jax.experimental.pallas import tpu as pltpu
from jax.experimental.pallas import tpu_sc as plsc

_NC = 2
_NS = 16
_NW = _NC * _NS
_CHUNK = 128


@functools.cache
def _make_lookup(B: int, D: int):
    assert B % (_NW * _CHUNK) == 0
    b_per_w = B // _NW
    n_chunks = b_per_w // _CHUNK
    mesh = plsc.VectorSubcoreMesh(core_axis_name="c", subcore_axis_name="s")

    @functools.partial(
        pl.kernel,
        mesh=mesh,
        out_type=jax.ShapeDtypeStruct((B, D), jnp.float32),
        scratch_types=[
            pltpu.VMEM((_CHUNK,), jnp.int32),
            pltpu.VMEM((_CHUNK, D), jnp.float32),
            pltpu.SemaphoreType.DMA,
        ],
        compiler_params=pltpu.CompilerParams(use_tc_tiling_on_sc=False),
    )
    def lookup(idx_hbm, table_hbm, out_hbm, idx_v, rows_v, sem):
        wid = lax.axis_index("s") * _NC + lax.axis_index("c")
        base = wid * b_per_w

        def step(i, carry):
            off = base + i * _CHUNK
            pltpu.sync_copy(idx_hbm.at[pl.ds(off, _CHUNK)], idx_v)
            pltpu.async_copy(table_hbm.at[idx_v], rows_v, sem).wait()
            pltpu.sync_copy(rows_v, out_hbm.at[pl.ds(off, _CHUNK)])
            return carry

        lax.fori_loop(0, n_chunks, step, 0)

    return lookup


def kernel(token_ids, weight):
    bsz, seq = token_ids.shape
    idx = token_ids.reshape(-1).astype(jnp.int32)
    out = _make_lookup(idx.shape[0], weight.shape[1])(idx, weight)
    return out.reshape(bsz, seq, weight.shape[1])


# speedup vs baseline: 1.1786x; 1.1786x over previous
import functools

import jax
import jax.numpy as jnp
from jax import lax
from jax.experimental import pallas as pl
from jax.experimental.pallas import tpu as pltpu
from jax.experimental.pallas import tpu_sc as plsc

_NC = 2
_NS = 16
_NW = _NC * _NS
_IW = 128
_K = 4
_SUP = _K * _IW


@functools.cache
def _make_lookup(B: int, D: int):
    assert B % (_NW * _SUP) == 0
    b_per_w = B // _NW
    n_sup = b_per_w // _SUP
    assert n_sup % 2 == 0
    mesh = plsc.VectorSubcoreMesh(core_axis_name="c", subcore_axis_name="s")

    @functools.partial(
        pl.kernel,
        mesh=mesh,
        out_type=jax.ShapeDtypeStruct((B, D), jnp.float32),
        scratch_types=[
            pltpu.VMEM((2, _K, _IW), jnp.int32),
            pltpu.VMEM((2, _SUP, D), jnp.float32),
            pltpu.SemaphoreType.DMA,
            pltpu.SemaphoreType.DMA,
            pltpu.SemaphoreType.DMA,
            pltpu.SemaphoreType.DMA,
        ],
        compiler_params=pltpu.CompilerParams(use_tc_tiling_on_sc=False),
    )
    def lookup(idx_hbm, table_hbm, out_hbm, idx_v, rows_v, sg0, sg1, so0, so1):
        wid = lax.axis_index("s") * _NC + lax.axis_index("c")
        base = wid * b_per_w
        irow = wid * (b_per_w // _IW)
        sg = (sg0, sg1)
        so = (so0, so1)

        def load_and_fire(g, b):
            pltpu.sync_copy(idx_hbm.at[pl.ds(irow + g * _K, _K)], idx_v.at[b])
            for j in range(_K):
                pltpu.make_async_copy(
                    table_hbm.at[idx_v.at[b].at[j]],
                    rows_v.at[b].at[pl.ds(j * _IW, _IW)],
                    sg[b],
                ).start()

        def drain_and_writeback(g, b):
            pltpu.make_async_copy(
                out_hbm.at[pl.ds(base + g * _SUP, _SUP)], rows_v.at[b], sg[b]
            ).wait()
            pltpu.make_async_copy(
                rows_v.at[b], out_hbm.at[pl.ds(base + g * _SUP, _SUP)], so[b]
            ).start()

        def wait_writeback(g, b):
            pltpu.make_async_copy(
                rows_v.at[b], out_hbm.at[pl.ds(base + g * _SUP, _SUP)], so[b]
            ).wait()

        load_and_fire(0, 0)
        load_and_fire(1, 1)
        drain_and_writeback(0, 0)

        def body(i, carry):
            g0 = 2 * i
            g1 = g0 + 1
            wait_writeback(g0 - 2, 0)
            load_and_fire(g0, 0)
            drain_and_writeback(g0 - 1, 1)
            wait_writeback(g1 - 2, 1)
            load_and_fire(g1, 1)
            drain_and_writeback(g0, 0)
            return carry

        lax.fori_loop(1, n_sup // 2, body, 0)

        wait_writeback(n_sup - 2, 0)
        drain_and_writeback(n_sup - 1, 1)
        wait_writeback(n_sup - 1, 1)

    return lookup


def kernel(token_ids, weight):
    bsz, seq = token_ids.shape
    idx = token_ids.reshape(-1, _IW).astype(jnp.int32)
    out = _make_lookup(idx.size, weight.shape[1])(idx, weight)
    return out.reshape(bsz, seq, weight.shape[1])


# speedup vs baseline: 1.1854x; 1.0058x over previous
import functools

import jax
import jax.numpy as jnp
from jax import lax
from jax.experimental import pallas as pl
from jax.experimental.pallas import tpu as pltpu
from jax.experimental.pallas import tpu_sc as plsc

_NC = 2
_NS = 16
_NW = _NC * _NS
_R = 4


@functools.cache
def _make_lookup(BT: int, L: int, D: int):
    rows_per_w = BT // _NW
    n_sup = rows_per_w // _R
    assert n_sup % 2 == 0
    mesh = plsc.VectorSubcoreMesh(core_axis_name="c", subcore_axis_name="s")

    @functools.partial(
        pl.kernel,
        mesh=mesh,
        out_type=jax.ShapeDtypeStruct((BT, L, D), jnp.float32),
        scratch_types=[
            pltpu.VMEM((2, _R, L), jnp.int32),
            pltpu.VMEM((2, _R, L, D), jnp.float32),
            pltpu.SemaphoreType.DMA,
            pltpu.SemaphoreType.DMA,
            pltpu.SemaphoreType.DMA,
            pltpu.SemaphoreType.DMA,
        ],
        compiler_params=pltpu.CompilerParams(use_tc_tiling_on_sc=False),
    )
    def lookup(idx_hbm, table_hbm, out_hbm, idx_v, rows_v, sg0, sg1, so0, so1):
        wid = lax.axis_index("s") * _NC + lax.axis_index("c")
        row0 = wid * rows_per_w
        sg = (sg0, sg1)
        so = (so0, so1)

        def load_and_fire(g, b):
            pltpu.sync_copy(idx_hbm.at[pl.ds(row0 + g * _R, _R)], idx_v.at[b])
            for j in range(_R):
                pltpu.make_async_copy(
                    table_hbm.at[idx_v.at[b].at[j]],
                    rows_v.at[b].at[j],
                    sg[b],
                ).start()

        def drain_and_writeback(g, b):
            pltpu.make_async_copy(
                out_hbm.at[pl.ds(row0 + g * _R, _R)], rows_v.at[b], sg[b]
            ).wait()
            pltpu.make_async_copy(
                rows_v.at[b], out_hbm.at[pl.ds(row0 + g * _R, _R)], so[b]
            ).start()

        def wait_writeback(g, b):
            pltpu.make_async_copy(
                rows_v.at[b], out_hbm.at[pl.ds(row0 + g * _R, _R)], so[b]
            ).wait()

        load_and_fire(0, 0)
        load_and_fire(1, 1)
        drain_and_writeback(0, 0)

        def body(i, carry):
            g0 = 2 * i
            g1 = g0 + 1
            wait_writeback(g0 - 2, 0)
            load_and_fire(g0, 0)
            drain_and_writeback(g0 - 1, 1)
            wait_writeback(g1 - 2, 1)
            load_and_fire(g1, 1)
            drain_and_writeback(g0, 0)
            return carry

        lax.fori_loop(1, n_sup // 2, body, 0)

        wait_writeback(n_sup - 2, 0)
        drain_and_writeback(n_sup - 1, 1)
        wait_writeback(n_sup - 1, 1)

    return lookup


def kernel(token_ids, weight):
    bsz, seq = token_ids.shape
    out = _make_lookup(bsz, seq, weight.shape[1])(
        token_ids.astype(jnp.int32), weight
    )
    return out


# speedup vs baseline: 1.4527x; 1.2254x over previous
import functools

import jax
import jax.numpy as jnp
from jax import lax
from jax.experimental import pallas as pl
from jax.experimental.pallas import tpu as pltpu
from jax.experimental.pallas import tpu_sc as plsc

_NC = 2
_NS = 16
_NW = _NC * _NS
_IW = 128
_K = 2
_SUP = _K * _IW


@functools.cache
def _make_lookup(B: int, DP: int):
    assert B % (_NW * _SUP) == 0
    b_per_w = B // _NW
    n_sup = b_per_w // _SUP
    assert n_sup % 2 == 0
    mesh = plsc.VectorSubcoreMesh(core_axis_name="c", subcore_axis_name="s")

    @functools.partial(
        pl.kernel,
        mesh=mesh,
        out_type=jax.ShapeDtypeStruct((B, DP), jnp.float32),
        scratch_types=[
            pltpu.VMEM((2, _K, _IW), jnp.int32),
            pltpu.VMEM((2, _SUP, DP), jnp.float32),
            pltpu.SemaphoreType.DMA,
            pltpu.SemaphoreType.DMA,
            pltpu.SemaphoreType.DMA,
            pltpu.SemaphoreType.DMA,
        ],
        compiler_params=pltpu.CompilerParams(use_tc_tiling_on_sc=True),
    )
    def lookup(idx_hbm, table_hbm, out_hbm, idx_v, rows_v, sg0, sg1, so0, so1):
        wid = lax.axis_index("s") * _NC + lax.axis_index("c")
        base = wid * b_per_w
        irow = wid * (b_per_w // _IW)
        sg = (sg0, sg1)
        so = (so0, so1)

        def load_and_fire(g, b):
            pltpu.sync_copy(idx_hbm.at[pl.ds(irow + g * _K, _K)], idx_v.at[b])
            for j in range(_K):
                pltpu.make_async_copy(
                    table_hbm.at[idx_v.at[b].at[j]],
                    rows_v.at[b].at[pl.ds(j * _IW, _IW)],
                    sg[b],
                ).start()

        def drain_and_writeback(g, b):
            pltpu.make_async_copy(
                out_hbm.at[pl.ds(base + g * _SUP, _SUP)], rows_v.at[b], sg[b]
            ).wait()
            pltpu.make_async_copy(
                rows_v.at[b], out_hbm.at[pl.ds(base + g * _SUP, _SUP)], so[b]
            ).start()

        def wait_writeback(g, b):
            pltpu.make_async_copy(
                rows_v.at[b], out_hbm.at[pl.ds(base + g * _SUP, _SUP)], so[b]
            ).wait()

        load_and_fire(0, 0)
        load_and_fire(1, 1)
        drain_and_writeback(0, 0)

        def body(i, carry):
            g0 = 2 * i
            g1 = g0 + 1
            wait_writeback(g0 - 2, 0)
            load_and_fire(g0, 0)
            drain_and_writeback(g0 - 1, 1)
            wait_writeback(g1 - 2, 1)
            load_and_fire(g1, 1)
            drain_and_writeback(g0, 0)
            return carry

        lax.fori_loop(1, n_sup // 2, body, 0)

        wait_writeback(n_sup - 2, 0)
        drain_and_writeback(n_sup - 1, 1)
        wait_writeback(n_sup - 1, 1)

    return lookup


def kernel(token_ids, weight):
    bsz, seq = token_ids.shape
    n, d = weight.shape
    idx2d = token_ids.reshape(-1, _IW).astype(jnp.int32)
    w_pad = jnp.pad(weight, ((0, 0), (0, _IW - d)))
    out = _make_lookup(bsz * seq, _IW)(idx2d, w_pad)
    return out[:, :d].reshape(bsz, seq, d)
